# Initial kernel scaffold; baseline (speedup 1.0000x reference)
import functools

import jax
import jax.numpy as jnp
from jax import lax
from jax.experimental import pallas as pl
from jax.experimental.pallas import tpu as pltpu
import jax.experimental.pallas.tpu_sc as plsc

NC = 2
NS = 16
CHUNK = 128
BN = 1000
EPS = 1e-5


def _edge_partials(z, src, dst):
  n, d = z.shape
  e = src.shape[0]
  assert e % CHUNK == 0 and n % NS == 0
  n_chunks = e // CHUNK
  nw = NC * NS
  chunks_per_tile = -(-n_chunks // nw)
  rows_per_tile = n // NS

  mesh = plsc.VectorSubcoreMesh(core_axis_name="c", subcore_axis_name="s")

  @functools.partial(
      pl.kernel,
      out_type=jax.ShapeDtypeStruct((NC, n, d), jnp.float32),
      mesh=mesh,
      scratch_types=[
          pltpu.VMEM((CHUNK,), jnp.int32),
          pltpu.VMEM((CHUNK,), jnp.int32),
          pltpu.VMEM((CHUNK, d), jnp.float32),
          pltpu.VMEM_SHARED((n, d), jnp.float32),
          pltpu.SemaphoreType.DMA,
      ],
  )
  def agg(z_hbm, src_hbm, dst_hbm, out_hbm, src_v, dst_v, rows_v, acc_sh, sem):
    c = lax.axis_index("c")
    s = lax.axis_index("s")
    w = s * NC + c
    r0 = s * rows_per_tile
    pltpu.sync_copy(z_hbm.at[pl.ds(r0, rows_per_tile)],
                    acc_sh.at[pl.ds(r0, rows_per_tile)])
    plsc.subcore_barrier()

    def body(i, carry):
      chunk = i * nw + w

      @pl.when(chunk < n_chunks)
      def _():
        base = chunk * CHUNK
        pltpu.sync_copy(src_hbm.at[pl.ds(base, CHUNK)], src_v)
        pltpu.sync_copy(dst_hbm.at[pl.ds(base, CHUNK)], dst_v)
        pltpu.async_copy(z_hbm.at[src_v], rows_v, sem).wait()
        pltpu.sync_copy(rows_v, acc_sh.at[dst_v], add=True)

      return carry

    lax.fori_loop(0, chunks_per_tile, body, 0)
    plsc.subcore_barrier()
    pltpu.sync_copy(acc_sh.at[pl.ds(r0, rows_per_tile)],
                    out_hbm.at[c, pl.ds(r0, rows_per_tile)])

  return agg(z, src, dst)


def _mlp_body(z_ref, p0_ref, p1_ref, w1_ref, b1_ref, w2_ref, b2_ref,
              t_ref, sums_ref):
  i = pl.program_id(0)
  h = p0_ref[...] + p1_ref[...] - z_ref[...]
  h = jnp.maximum(
      jnp.dot(h, w1_ref[...], preferred_element_type=jnp.float32) + b1_ref[...],
      0.0)
  h = jnp.dot(h, w2_ref[...], preferred_element_type=jnp.float32) + b2_ref[...]
  t = jnp.maximum(h, 0.0)
  t_ref[...] = t

  @pl.when(i == 0)
  def _():
    sums_ref[...] = jnp.zeros_like(sums_ref)

  sums_ref[...] += jnp.concatenate(
      [jnp.sum(t, axis=0, keepdims=True),
       jnp.sum(t * t, axis=0, keepdims=True)], axis=0)


def _mlp_pass(z, p0, p1, w1, b1, w2, b2):
  n, h = z.shape[0], w1.shape[1]
  nb = n // BN
  blk = lambda i: (i, 0)
  full = lambda i: (0, 0)
  return pl.pallas_call(
      _mlp_body,
      grid=(nb,),
      in_specs=[
          pl.BlockSpec((BN, z.shape[1]), blk),
          pl.BlockSpec((BN, z.shape[1]), blk),
          pl.BlockSpec((BN, z.shape[1]), blk),
          pl.BlockSpec(w1.shape, full),
          pl.BlockSpec((1, h), full),
          pl.BlockSpec(w2.shape, full),
          pl.BlockSpec((1, h), full),
      ],
      out_specs=[
          pl.BlockSpec((BN, h), blk),
          pl.BlockSpec((2, h), full),
      ],
      out_shape=[
          jax.ShapeDtypeStruct((n, h), jnp.float32),
          jax.ShapeDtypeStruct((2, h), jnp.float32),
      ],
  )(z, p0, p1, w1, b1.reshape(1, -1), w2, b2.reshape(1, -1))


def _bn_pool_body(t_ref, sums_ref, gamma_ref, beta_ref, batch_ref,
                  h_ref, g_ref, *, n_nodes, n_graphs):
  i = pl.program_id(0)
  inv_n = 1.0 / n_nodes
  mu = sums_ref[0:1, :] * inv_n
  var = jnp.maximum(sums_ref[1:2, :] * inv_n - mu * mu, 0.0)
  scale = gamma_ref[...] * lax.rsqrt(var + EPS)
  shift = beta_ref[...] - mu * scale
  h = t_ref[...] * scale + shift
  h_ref[...] = h

  bb = batch_ref[0]
  gi = lax.broadcasted_iota(jnp.int32, (n_graphs, h.shape[0]), 0)
  onehot = (gi == bb).astype(jnp.float32)

  @pl.when(i == 0)
  def _():
    g_ref[...] = jnp.zeros_like(g_ref)

  g_ref[...] += jnp.dot(onehot, h, preferred_element_type=jnp.float32)


def _bn_pool_pass(t, sums, gamma, beta, batch3d, n_graphs):
  n, h = t.shape
  nb = n // BN
  return pl.pallas_call(
      functools.partial(_bn_pool_body, n_nodes=n, n_graphs=n_graphs),
      grid=(nb,),
      in_specs=[
          pl.BlockSpec((BN, h), lambda i: (i, 0)),
          pl.BlockSpec((2, h), lambda i: (0, 0)),
          pl.BlockSpec((1, h), lambda i: (0, 0)),
          pl.BlockSpec((1, h), lambda i: (0, 0)),
          pl.BlockSpec((1, 1, BN), lambda i: (i, 0, 0)),
      ],
      out_specs=[
          pl.BlockSpec((BN, h), lambda i: (i, 0)),
          pl.BlockSpec((n_graphs, h), lambda i: (0, 0)),
      ],
      out_shape=[
          jax.ShapeDtypeStruct((n, h), jnp.float32),
          jax.ShapeDtypeStruct((n_graphs, h), jnp.float32),
      ],
  )(t, sums, gamma.reshape(1, -1), beta.reshape(1, -1), batch3d)


def kernel(x, edge_index, batch,
           W1_0, b1_0, W2_0, b2_0, gamma_0, beta_0,
           W1_1, b1_1, W2_1, b2_1, gamma_1, beta_1):
  n = x.shape[0]
  g_count = 512
  src = edge_index[0]
  dst = edge_index[1]
  batch3d = batch.reshape(n // BN, 1, BN)

  params = [(W1_0, b1_0, W2_0, b2_0, gamma_0, beta_0),
            (W1_1, b1_1, W2_1, b2_1, gamma_1, beta_1)]
  z = x
  hs, gs = [], []
  for (w1, b1, w2, b2, ga, be) in params:
    partials = _edge_partials(z, src, dst)
    t, sums = _mlp_pass(z, partials[0], partials[1], w1, b1, w2, b2)
    h, g = _bn_pool_pass(t, sums, ga, be, batch3d, g_count)
    hs.append(h)
    gs.append(g)
    z = h
  return jnp.concatenate(hs, axis=1), jnp.concatenate(gs, axis=1)


# speedup vs baseline: 5.5860x; 5.5860x over previous
import functools

import jax
import jax.numpy as jnp
from jax import lax
from jax.experimental import pallas as pl
from jax.experimental.pallas import tpu as pltpu
import jax.experimental.pallas.tpu_sc as plsc

NC = 2
NS = 16
CHUNK = 128
BN = 1000
EPS = 1e-5


def _edge_partials(z, src, dst):
  n, d = z.shape
  e = src.shape[0]
  assert e % CHUNK == 0 and n % NS == 0
  n_chunks = e // CHUNK
  nw = NC * NS
  chunks_per_tile = -(-n_chunks // nw)
  rows_a = 640
  rows_last = n - (NS - 1) * rows_a
  assert rows_last > 0 and rows_last % 8 == 0

  mesh = plsc.VectorSubcoreMesh(core_axis_name="c", subcore_axis_name="s")

  @functools.partial(
      pl.kernel,
      out_type=jax.ShapeDtypeStruct((NC, n, d), jnp.float32),
      mesh=mesh,
      scratch_types=[
          pltpu.VMEM((CHUNK,), jnp.int32),
          pltpu.VMEM((CHUNK,), jnp.int32),
          pltpu.VMEM((CHUNK, d), jnp.float32),
          pltpu.VMEM_SHARED((n, d), jnp.float32),
          pltpu.SemaphoreType.DMA,
      ],
  )
  def agg(z_hbm, src_hbm, dst_hbm, out_hbm, src_v, dst_v, rows_v, acc_sh, sem):
    c = lax.axis_index("c")
    s = lax.axis_index("s")
    w = s * NC + c
    r0 = s * rows_a

    @pl.when(s < NS - 1)
    def _():
      pltpu.sync_copy(z_hbm.at[pl.ds(r0, rows_a)],
                      acc_sh.at[pl.ds(r0, rows_a)])

    @pl.when(s == NS - 1)
    def _():
      pltpu.sync_copy(z_hbm.at[pl.ds(r0, rows_last)],
                      acc_sh.at[pl.ds(r0, rows_last)])

    plsc.subcore_barrier()

    def body(i, carry):
      chunk = i * nw + w

      @pl.when(chunk < n_chunks)
      def _():
        base = chunk * CHUNK
        pltpu.sync_copy(src_hbm.at[pl.ds(base, CHUNK)], src_v)
        pltpu.sync_copy(dst_hbm.at[pl.ds(base, CHUNK)], dst_v)
        pltpu.async_copy(z_hbm.at[src_v], rows_v, sem).wait()
        pltpu.sync_copy(rows_v, acc_sh.at[dst_v], add=True)

      return carry

    lax.fori_loop(0, chunks_per_tile, body, 0)
    plsc.subcore_barrier()

    @pl.when(s < NS - 1)
    def _():
      pltpu.sync_copy(acc_sh.at[pl.ds(r0, rows_a)],
                      out_hbm.at[c, pl.ds(r0, rows_a)])

    @pl.when(s == NS - 1)
    def _():
      pltpu.sync_copy(acc_sh.at[pl.ds(r0, rows_last)],
                      out_hbm.at[c, pl.ds(r0, rows_last)])

  return agg(z, src, dst)


def _mlp_body(z_ref, p0_ref, p1_ref, w1_ref, b1_ref, w2_ref, b2_ref,
              t_ref, sums_ref):
  i = pl.program_id(0)
  h = p0_ref[...] + p1_ref[...] - z_ref[...]
  h = jnp.maximum(
      jnp.dot(h, w1_ref[...], preferred_element_type=jnp.float32) + b1_ref[...],
      0.0)
  h = jnp.dot(h, w2_ref[...], preferred_element_type=jnp.float32) + b2_ref[...]
  t = jnp.maximum(h, 0.0)
  t_ref[...] = t

  @pl.when(i == 0)
  def _():
    sums_ref[...] = jnp.zeros_like(sums_ref)

  sums_ref[...] += jnp.concatenate(
      [jnp.sum(t, axis=0, keepdims=True),
       jnp.sum(t * t, axis=0, keepdims=True)], axis=0)


def _mlp_pass(z, p0, p1, w1, b1, w2, b2):
  n, h = z.shape[0], w1.shape[1]
  nb = n // BN
  blk = lambda i: (i, 0)
  full = lambda i: (0, 0)
  return pl.pallas_call(
      _mlp_body,
      grid=(nb,),
      in_specs=[
          pl.BlockSpec((BN, z.shape[1]), blk),
          pl.BlockSpec((BN, z.shape[1]), blk),
          pl.BlockSpec((BN, z.shape[1]), blk),
          pl.BlockSpec(w1.shape, full),
          pl.BlockSpec((1, h), full),
          pl.BlockSpec(w2.shape, full),
          pl.BlockSpec((1, h), full),
      ],
      out_specs=[
          pl.BlockSpec((BN, h), blk),
          pl.BlockSpec((2, h), full),
      ],
      out_shape=[
          jax.ShapeDtypeStruct((n, h), jnp.float32),
          jax.ShapeDtypeStruct((2, h), jnp.float32),
      ],
  )(z, p0, p1, w1, b1.reshape(1, -1), w2, b2.reshape(1, -1))


def _bn_pool_body(t_ref, sums_ref, gamma_ref, beta_ref, batch_ref,
                  h_ref, g_ref, *, n_nodes, n_graphs):
  i = pl.program_id(0)
  inv_n = 1.0 / n_nodes
  mu = sums_ref[0:1, :] * inv_n
  var = jnp.maximum(sums_ref[1:2, :] * inv_n - mu * mu, 0.0)
  scale = gamma_ref[...] * lax.rsqrt(var + EPS)
  shift = beta_ref[...] - mu * scale
  h = t_ref[...] * scale + shift
  h_ref[...] = h

  bb = batch_ref[0]
  gi = lax.broadcasted_iota(jnp.int32, (n_graphs, h.shape[0]), 0)
  onehot = (gi == bb).astype(jnp.float32)

  @pl.when(i == 0)
  def _():
    g_ref[...] = jnp.zeros_like(g_ref)

  g_ref[...] += jnp.dot(onehot, h, preferred_element_type=jnp.float32)


def _bn_pool_pass(t, sums, gamma, beta, batch3d, n_graphs):
  n, h = t.shape
  nb = n // BN
  return pl.pallas_call(
      functools.partial(_bn_pool_body, n_nodes=n, n_graphs=n_graphs),
      grid=(nb,),
      in_specs=[
          pl.BlockSpec((BN, h), lambda i: (i, 0)),
          pl.BlockSpec((2, h), lambda i: (0, 0)),
          pl.BlockSpec((1, h), lambda i: (0, 0)),
          pl.BlockSpec((1, h), lambda i: (0, 0)),
          pl.BlockSpec((1, 1, BN), lambda i: (i, 0, 0)),
      ],
      out_specs=[
          pl.BlockSpec((BN, h), lambda i: (i, 0)),
          pl.BlockSpec((n_graphs, h), lambda i: (0, 0)),
      ],
      out_shape=[
          jax.ShapeDtypeStruct((n, h), jnp.float32),
          jax.ShapeDtypeStruct((n_graphs, h), jnp.float32),
      ],
  )(t, sums, gamma.reshape(1, -1), beta.reshape(1, -1), batch3d)


def kernel(x, edge_index, batch,
           W1_0, b1_0, W2_0, b2_0, gamma_0, beta_0,
           W1_1, b1_1, W2_1, b2_1, gamma_1, beta_1):
  n = x.shape[0]
  g_count = 512
  src = edge_index[0]
  dst = edge_index[1]
  batch3d = batch.reshape(n // BN, 1, BN)

  params = [(W1_0, b1_0, W2_0, b2_0, gamma_0, beta_0),
            (W1_1, b1_1, W2_1, b2_1, gamma_1, beta_1)]
  z = x
  hs, gs = [], []
  for (w1, b1, w2, b2, ga, be) in params:
    partials = _edge_partials(z, src, dst)
    t, sums = _mlp_pass(z, partials[0], partials[1], w1, b1, w2, b2)
    h, g = _bn_pool_pass(t, sums, ga, be, batch3d, g_count)
    hs.append(h)
    gs.append(g)
    z = h
  return jnp.concatenate(hs, axis=1), jnp.concatenate(gs, axis=1)


# speedup vs baseline: 10.5546x; 1.8895x over previous
import functools

import jax
import jax.numpy as jnp
from jax import lax
from jax.experimental import pallas as pl
from jax.experimental.pallas import tpu as pltpu
import jax.experimental.pallas.tpu_sc as plsc

NC = 2
NS = 16
CHUNK = 128
BN = 1000
EPS = 1e-5


def _edge_partials(z, src, dst):
  n, d = z.shape
  e = src.shape[0]
  assert e % CHUNK == 0 and n % NS == 0
  n_chunks = e // CHUNK
  nw = NC * NS
  chunks_per_tile = -(-n_chunks // nw)
  rows_a = 640
  rows_last = n - (NS - 1) * rows_a
  assert rows_last > 0 and rows_last % 8 == 0

  mesh = plsc.VectorSubcoreMesh(core_axis_name="c", subcore_axis_name="s")

  n_slots = chunks_per_tile
  assert n_slots >= 4

  @functools.partial(
      pl.kernel,
      out_type=jax.ShapeDtypeStruct((NC, n, d), jnp.float32),
      mesh=mesh,
      scratch_types=[
          [pltpu.VMEM((CHUNK,), jnp.int32) for _ in range(4)],
          [pltpu.VMEM((CHUNK,), jnp.int32) for _ in range(4)],
          [pltpu.VMEM((CHUNK, d), jnp.float32) for _ in range(2)],
          pltpu.VMEM_SHARED((n, d), jnp.float32),
          [pltpu.SemaphoreType.DMA for _ in range(4)],
          [pltpu.SemaphoreType.DMA for _ in range(4)],
          [pltpu.SemaphoreType.DMA for _ in range(2)],
          [pltpu.SemaphoreType.DMA for _ in range(2)],
      ],
  )
  def agg(z_hbm, src_hbm, dst_hbm, out_hbm, src_v, dst_v, rows_v, acc_sh,
          sem_si, sem_di, sem_g, sem_s):
    c = lax.axis_index("c")
    s = lax.axis_index("s")
    w = s * NC + c
    r0 = s * rows_a

    @pl.when(s < NS - 1)
    def _():
      pltpu.sync_copy(z_hbm.at[pl.ds(r0, rows_a)],
                      acc_sh.at[pl.ds(r0, rows_a)])

    @pl.when(s == NS - 1)
    def _():
      pltpu.sync_copy(z_hbm.at[pl.ds(r0, rows_last)],
                      acc_sh.at[pl.ds(r0, rows_last)])

    plsc.subcore_barrier()

    def valid(slot):
      return (slot * nw + w) < n_chunks

    def stage_a(slot, ib):
      @pl.when(valid(slot))
      def _():
        base = (slot * nw + w) * CHUNK
        pltpu.async_copy(src_hbm.at[pl.ds(base, CHUNK)], src_v[ib], sem_si[ib])
        pltpu.async_copy(dst_hbm.at[pl.ds(base, CHUNK)], dst_v[ib], sem_di[ib])

    def stage_b(slot, ib, rb):
      @pl.when(valid(slot))
      def _():
        base = (slot * nw + w) * CHUNK
        pltpu.make_async_copy(src_hbm.at[pl.ds(base, CHUNK)], src_v[ib],
                              sem_si[ib]).wait()
        pltpu.async_copy(z_hbm.at[src_v[ib]], rows_v[rb], sem_g[rb])

    def stage_c(slot, ib, rb):
      @pl.when(valid(slot))
      def _():
        base = (slot * nw + w) * CHUNK
        pltpu.make_async_copy(z_hbm.at[src_v[ib]], rows_v[rb],
                              sem_g[rb]).wait()
        pltpu.make_async_copy(dst_hbm.at[pl.ds(base, CHUNK)], dst_v[ib],
                              sem_di[ib]).wait()
        pltpu.async_copy(rows_v[rb], acc_sh.at[dst_v[ib]], sem_s[rb],
                         add=True)

    def stage_w(slot, ib, rb):
      @pl.when(valid(slot))
      def _():
        pltpu.make_async_copy(rows_v[rb], acc_sh.at[dst_v[ib]],
                              sem_s[rb]).wait()

    def step(slot, k):
      if not isinstance(slot, int) or slot - 1 >= 0:
        stage_w(slot - 1, (k - 1) % 4, (k - 1) % 2)
      stage_a(slot + 3, (k + 3) % 4)
      stage_b(slot + 1, (k + 1) % 4, (k + 1) % 2)
      stage_c(slot, k % 4, k % 2)

    stage_a(0, 0)
    stage_a(1, 1)
    stage_a(2, 2)
    stage_b(0, 0, 0)
    step(0, 0)
    step(1, 1)

    n_main = (n_slots - 4) // 4

    def body(j, carry):
      s0 = 2 + 4 * j
      for k in range(4):
        step(s0 + k, 2 + k)
      return carry

    lax.fori_loop(0, n_main, body, 0)
    for tail in range(2 + 4 * n_main, n_slots):
      step(tail, tail % 4)
    stage_w(n_slots - 1, (n_slots - 1) % 4, (n_slots - 1) % 2)

    plsc.subcore_barrier()

    @pl.when(s < NS - 1)
    def _():
      pltpu.sync_copy(acc_sh.at[pl.ds(r0, rows_a)],
                      out_hbm.at[c, pl.ds(r0, rows_a)])

    @pl.when(s == NS - 1)
    def _():
      pltpu.sync_copy(acc_sh.at[pl.ds(r0, rows_last)],
                      out_hbm.at[c, pl.ds(r0, rows_last)])

  return agg(z, src, dst)


def _mlp_body(z_ref, p0_ref, p1_ref, w1_ref, b1_ref, w2_ref, b2_ref,
              t_ref, sums_ref):
  i = pl.program_id(0)
  h = p0_ref[...] + p1_ref[...] - z_ref[...]
  h = jnp.maximum(
      jnp.dot(h, w1_ref[...], preferred_element_type=jnp.float32) + b1_ref[...],
      0.0)
  h = jnp.dot(h, w2_ref[...], preferred_element_type=jnp.float32) + b2_ref[...]
  t = jnp.maximum(h, 0.0)
  t_ref[...] = t

  @pl.when(i == 0)
  def _():
    sums_ref[...] = jnp.zeros_like(sums_ref)

  sums_ref[...] += jnp.concatenate(
      [jnp.sum(t, axis=0, keepdims=True),
       jnp.sum(t * t, axis=0, keepdims=True)], axis=0)


def _mlp_pass(z, p0, p1, w1, b1, w2, b2):
  n, h = z.shape[0], w1.shape[1]
  nb = n // BN
  blk = lambda i: (i, 0)
  full = lambda i: (0, 0)
  return pl.pallas_call(
      _mlp_body,
      grid=(nb,),
      in_specs=[
          pl.BlockSpec((BN, z.shape[1]), blk),
          pl.BlockSpec((BN, z.shape[1]), blk),
          pl.BlockSpec((BN, z.shape[1]), blk),
          pl.BlockSpec(w1.shape, full),
          pl.BlockSpec((1, h), full),
          pl.BlockSpec(w2.shape, full),
          pl.BlockSpec((1, h), full),
      ],
      out_specs=[
          pl.BlockSpec((BN, h), blk),
          pl.BlockSpec((2, h), full),
      ],
      out_shape=[
          jax.ShapeDtypeStruct((n, h), jnp.float32),
          jax.ShapeDtypeStruct((2, h), jnp.float32),
      ],
  )(z, p0, p1, w1, b1.reshape(1, -1), w2, b2.reshape(1, -1))


def _bn_pool_body(t_ref, sums_ref, gamma_ref, beta_ref, batch_ref,
                  h_ref, g_ref, *, n_nodes, n_graphs):
  i = pl.program_id(0)
  inv_n = 1.0 / n_nodes
  mu = sums_ref[0:1, :] * inv_n
  var = jnp.maximum(sums_ref[1:2, :] * inv_n - mu * mu, 0.0)
  scale = gamma_ref[...] * lax.rsqrt(var + EPS)
  shift = beta_ref[...] - mu * scale
  h = t_ref[...] * scale + shift
  h_ref[...] = h

  bb = batch_ref[0]
  gi = lax.broadcasted_iota(jnp.int32, (n_graphs, h.shape[0]), 0)
  onehot = (gi == bb).astype(jnp.float32)

  @pl.when(i == 0)
  def _():
    g_ref[...] = jnp.zeros_like(g_ref)

  g_ref[...] += jnp.dot(onehot, h, preferred_element_type=jnp.float32)


def _bn_pool_pass(t, sums, gamma, beta, batch3d, n_graphs):
  n, h = t.shape
  nb = n // BN
  return pl.pallas_call(
      functools.partial(_bn_pool_body, n_nodes=n, n_graphs=n_graphs),
      grid=(nb,),
      in_specs=[
          pl.BlockSpec((BN, h), lambda i: (i, 0)),
          pl.BlockSpec((2, h), lambda i: (0, 0)),
          pl.BlockSpec((1, h), lambda i: (0, 0)),
          pl.BlockSpec((1, h), lambda i: (0, 0)),
          pl.BlockSpec((1, 1, BN), lambda i: (i, 0, 0)),
      ],
      out_specs=[
          pl.BlockSpec((BN, h), lambda i: (i, 0)),
          pl.BlockSpec((n_graphs, h), lambda i: (0, 0)),
      ],
      out_shape=[
          jax.ShapeDtypeStruct((n, h), jnp.float32),
          jax.ShapeDtypeStruct((n_graphs, h), jnp.float32),
      ],
  )(t, sums, gamma.reshape(1, -1), beta.reshape(1, -1), batch3d)


def kernel(x, edge_index, batch,
           W1_0, b1_0, W2_0, b2_0, gamma_0, beta_0,
           W1_1, b1_1, W2_1, b2_1, gamma_1, beta_1):
  n = x.shape[0]
  g_count = 512
  src = edge_index[0]
  dst = edge_index[1]
  batch3d = batch.reshape(n // BN, 1, BN)

  params = [(W1_0, b1_0, W2_0, b2_0, gamma_0, beta_0),
            (W1_1, b1_1, W2_1, b2_1, gamma_1, beta_1)]
  z = x
  hs, gs = [], []
  for (w1, b1, w2, b2, ga, be) in params:
    partials = _edge_partials(z, src, dst)
    t, sums = _mlp_pass(z, partials[0], partials[1], w1, b1, w2, b2)
    h, g = _bn_pool_pass(t, sums, ga, be, batch3d, g_count)
    hs.append(h)
    gs.append(g)
    z = h
  return jnp.concatenate(hs, axis=1), jnp.concatenate(gs, axis=1)


# speedup vs baseline: 10.9457x; 1.0371x over previous
import functools

import jax
import jax.numpy as jnp
from jax import lax
from jax.experimental import pallas as pl
from jax.experimental.pallas import tpu as pltpu
import jax.experimental.pallas.tpu_sc as plsc

NC = 2
NS = 16
CHUNK = 128
BN = 1000
EPS = 1e-5


def _edge_partials(z, src, dst):
  n, d = z.shape
  e = src.shape[0]
  assert e % CHUNK == 0 and n % NS == 0
  n_chunks = e // CHUNK
  nw = NC * NS
  chunks_per_tile = -(-n_chunks // nw)
  rows_a = 640
  rows_last = n - (NS - 1) * rows_a
  assert rows_last > 0 and rows_last % 8 == 0

  mesh = plsc.VectorSubcoreMesh(core_axis_name="c", subcore_axis_name="s")

  IDX_D, ROWS_D, LOOKA, PEEL = 6, 3, 4, 6
  n_slots = chunks_per_tile
  assert n_slots >= 2 * PEEL

  @functools.partial(
      pl.kernel,
      out_type=jax.ShapeDtypeStruct((NC, n, d), jnp.float32),
      mesh=mesh,
      scratch_types=[
          [pltpu.VMEM((CHUNK,), jnp.int32) for _ in range(IDX_D)],
          [pltpu.VMEM((CHUNK,), jnp.int32) for _ in range(IDX_D)],
          [pltpu.VMEM((CHUNK, d), jnp.float32) for _ in range(ROWS_D)],
          pltpu.VMEM_SHARED((n, d), jnp.float32),
          [pltpu.SemaphoreType.DMA for _ in range(IDX_D)],
          [pltpu.SemaphoreType.DMA for _ in range(IDX_D)],
          [pltpu.SemaphoreType.DMA for _ in range(ROWS_D)],
          [pltpu.SemaphoreType.DMA for _ in range(ROWS_D)],
      ],
  )
  def agg(z_hbm, src_hbm, dst_hbm, out_hbm, src_v, dst_v, rows_v, acc_sh,
          sem_si, sem_di, sem_g, sem_s):
    c = lax.axis_index("c")
    s = lax.axis_index("s")
    w = s * NC + c
    r0 = s * rows_a

    @pl.when(s < NS - 1)
    def _():
      pltpu.sync_copy(z_hbm.at[pl.ds(r0, rows_a)],
                      acc_sh.at[pl.ds(r0, rows_a)])

    @pl.when(s == NS - 1)
    def _():
      pltpu.sync_copy(z_hbm.at[pl.ds(r0, rows_last)],
                      acc_sh.at[pl.ds(r0, rows_last)])

    plsc.subcore_barrier()

    def valid(slot):
      return (slot * nw + w) < n_chunks

    def stage_a(slot, ib):
      @pl.when(valid(slot))
      def _():
        base = (slot * nw + w) * CHUNK
        pltpu.async_copy(src_hbm.at[pl.ds(base, CHUNK)], src_v[ib], sem_si[ib])
        pltpu.async_copy(dst_hbm.at[pl.ds(base, CHUNK)], dst_v[ib], sem_di[ib])

    def stage_b(slot, ib, rb):
      @pl.when(valid(slot))
      def _():
        base = (slot * nw + w) * CHUNK
        pltpu.make_async_copy(src_hbm.at[pl.ds(base, CHUNK)], src_v[ib],
                              sem_si[ib]).wait()
        pltpu.async_copy(z_hbm.at[src_v[ib]], rows_v[rb], sem_g[rb])

    def stage_c(slot, ib, rb):
      @pl.when(valid(slot))
      def _():
        base = (slot * nw + w) * CHUNK
        pltpu.make_async_copy(z_hbm.at[src_v[ib]], rows_v[rb],
                              sem_g[rb]).wait()
        pltpu.make_async_copy(dst_hbm.at[pl.ds(base, CHUNK)], dst_v[ib],
                              sem_di[ib]).wait()
        pltpu.async_copy(rows_v[rb], acc_sh.at[dst_v[ib]], sem_s[rb],
                         add=True)

    def stage_w(slot, ib, rb):
      @pl.when(valid(slot))
      def _():
        pltpu.make_async_copy(rows_v[rb], acc_sh.at[dst_v[ib]],
                              sem_s[rb]).wait()

    def step(slot, k):
      if not isinstance(slot, int) or slot - 2 >= 0:
        stage_w(slot - 2, (k - 2) % IDX_D, (k - 2) % ROWS_D)
      stage_a(slot + LOOKA, (k + LOOKA) % IDX_D)
      stage_b(slot + 1, (k + 1) % IDX_D, (k + 1) % ROWS_D)
      stage_c(slot, k % IDX_D, k % ROWS_D)

    for p in range(LOOKA):
      stage_a(p, p % IDX_D)
    stage_b(0, 0, 0)
    for p in range(PEEL):
      step(p, p % IDX_D)

    n_main = (n_slots - 2 * PEEL) // PEEL

    def body(j, carry):
      s0 = PEEL + PEEL * j
      for k in range(PEEL):
        step(s0 + k, k % IDX_D)
      return carry

    lax.fori_loop(0, n_main, body, 0)
    for tail in range(PEEL + PEEL * n_main, n_slots):
      step(tail, tail % IDX_D)
    stage_w(n_slots - 2, (n_slots - 2) % IDX_D, (n_slots - 2) % ROWS_D)
    stage_w(n_slots - 1, (n_slots - 1) % IDX_D, (n_slots - 1) % ROWS_D)

    plsc.subcore_barrier()

    @pl.when(s < NS - 1)
    def _():
      pltpu.sync_copy(acc_sh.at[pl.ds(r0, rows_a)],
                      out_hbm.at[c, pl.ds(r0, rows_a)])

    @pl.when(s == NS - 1)
    def _():
      pltpu.sync_copy(acc_sh.at[pl.ds(r0, rows_last)],
                      out_hbm.at[c, pl.ds(r0, rows_last)])

  return agg(z, src, dst)


def _mlp_body(z_ref, p0_ref, p1_ref, w1_ref, b1_ref, w2_ref, b2_ref,
              t_ref, sums_ref):
  i = pl.program_id(0)
  h = p0_ref[...] + p1_ref[...] - z_ref[...]
  h = jnp.maximum(
      jnp.dot(h, w1_ref[...], preferred_element_type=jnp.float32) + b1_ref[...],
      0.0)
  h = jnp.dot(h, w2_ref[...], preferred_element_type=jnp.float32) + b2_ref[...]
  t = jnp.maximum(h, 0.0)
  t_ref[...] = t

  @pl.when(i == 0)
  def _():
    sums_ref[...] = jnp.zeros_like(sums_ref)

  sums_ref[...] += jnp.concatenate(
      [jnp.sum(t, axis=0, keepdims=True),
       jnp.sum(t * t, axis=0, keepdims=True)], axis=0)


def _mlp_pass(z, p0, p1, w1, b1, w2, b2):
  n, h = z.shape[0], w1.shape[1]
  nb = n // BN
  blk = lambda i: (i, 0)
  full = lambda i: (0, 0)
  return pl.pallas_call(
      _mlp_body,
      grid=(nb,),
      in_specs=[
          pl.BlockSpec((BN, z.shape[1]), blk),
          pl.BlockSpec((BN, z.shape[1]), blk),
          pl.BlockSpec((BN, z.shape[1]), blk),
          pl.BlockSpec(w1.shape, full),
          pl.BlockSpec((1, h), full),
          pl.BlockSpec(w2.shape, full),
          pl.BlockSpec((1, h), full),
      ],
      out_specs=[
          pl.BlockSpec((BN, h), blk),
          pl.BlockSpec((2, h), full),
      ],
      out_shape=[
          jax.ShapeDtypeStruct((n, h), jnp.float32),
          jax.ShapeDtypeStruct((2, h), jnp.float32),
      ],
  )(z, p0, p1, w1, b1.reshape(1, -1), w2, b2.reshape(1, -1))


def _bn_pool_body(t_ref, sums_ref, gamma_ref, beta_ref, batch_ref,
                  h_ref, g_ref, *, n_nodes, n_graphs):
  i = pl.program_id(0)
  inv_n = 1.0 / n_nodes
  mu = sums_ref[0:1, :] * inv_n
  var = jnp.maximum(sums_ref[1:2, :] * inv_n - mu * mu, 0.0)
  scale = gamma_ref[...] * lax.rsqrt(var + EPS)
  shift = beta_ref[...] - mu * scale
  h = t_ref[...] * scale + shift
  h_ref[...] = h

  bb = batch_ref[0]
  gi = lax.broadcasted_iota(jnp.int32, (n_graphs, h.shape[0]), 0)
  onehot = (gi == bb).astype(jnp.float32)

  @pl.when(i == 0)
  def _():
    g_ref[...] = jnp.zeros_like(g_ref)

  g_ref[...] += jnp.dot(onehot, h, preferred_element_type=jnp.float32)


def _bn_pool_pass(t, sums, gamma, beta, batch3d, n_graphs):
  n, h = t.shape
  nb = n // BN
  return pl.pallas_call(
      functools.partial(_bn_pool_body, n_nodes=n, n_graphs=n_graphs),
      grid=(nb,),
      in_specs=[
          pl.BlockSpec((BN, h), lambda i: (i, 0)),
          pl.BlockSpec((2, h), lambda i: (0, 0)),
          pl.BlockSpec((1, h), lambda i: (0, 0)),
          pl.BlockSpec((1, h), lambda i: (0, 0)),
          pl.BlockSpec((1, 1, BN), lambda i: (i, 0, 0)),
      ],
      out_specs=[
          pl.BlockSpec((BN, h), lambda i: (i, 0)),
          pl.BlockSpec((n_graphs, h), lambda i: (0, 0)),
      ],
      out_shape=[
          jax.ShapeDtypeStruct((n, h), jnp.float32),
          jax.ShapeDtypeStruct((n_graphs, h), jnp.float32),
      ],
  )(t, sums, gamma.reshape(1, -1), beta.reshape(1, -1), batch3d)


def kernel(x, edge_index, batch,
           W1_0, b1_0, W2_0, b2_0, gamma_0, beta_0,
           W1_1, b1_1, W2_1, b2_1, gamma_1, beta_1):
  n = x.shape[0]
  g_count = 512
  src = edge_index[0]
  dst = edge_index[1]
  batch3d = batch.reshape(n // BN, 1, BN)

  params = [(W1_0, b1_0, W2_0, b2_0, gamma_0, beta_0),
            (W1_1, b1_1, W2_1, b2_1, gamma_1, beta_1)]
  z = x
  hs, gs = [], []
  for (w1, b1, w2, b2, ga, be) in params:
    partials = _edge_partials(z, src, dst)
    t, sums = _mlp_pass(z, partials[0], partials[1], w1, b1, w2, b2)
    h, g = _bn_pool_pass(t, sums, ga, be, batch3d, g_count)
    hs.append(h)
    gs.append(g)
    z = h
  return jnp.concatenate(hs, axis=1), jnp.concatenate(gs, axis=1)


# speedup vs baseline: 11.5936x; 1.0592x over previous
import functools

import jax
import jax.numpy as jnp
from jax import lax
from jax.experimental import pallas as pl
from jax.experimental.pallas import tpu as pltpu
import jax.experimental.pallas.tpu_sc as plsc

NC = 2
NS = 16
CHUNK = 128
BN = 1000
EPS = 1e-5


def _edge_partials(z, src, dst):
  n, d = z.shape
  e = src.shape[0]
  assert e % CHUNK == 0 and n % NS == 0
  n_chunks = e // CHUNK
  nw = NC * NS
  chunks_per_tile = -(-n_chunks // nw)
  rows_a = 640
  rows_last = n - (NS - 1) * rows_a
  assert rows_last > 0 and rows_last % 8 == 0

  mesh = plsc.VectorSubcoreMesh(core_axis_name="c", subcore_axis_name="s")

  IDX_D, ROWS_D, LOOKA, PEEL = 6, 3, 4, 6
  n_slots = chunks_per_tile
  assert n_slots >= 2 * PEEL

  @functools.partial(
      pl.kernel,
      out_type=jax.ShapeDtypeStruct((NC, n, d), jnp.float32),
      mesh=mesh,
      scratch_types=[
          [pltpu.VMEM((CHUNK,), jnp.int32) for _ in range(IDX_D)],
          [pltpu.VMEM((CHUNK,), jnp.int32) for _ in range(IDX_D)],
          [pltpu.VMEM((CHUNK, d), jnp.float32) for _ in range(ROWS_D)],
          pltpu.VMEM_SHARED((n, d), jnp.float32),
          [pltpu.SemaphoreType.DMA for _ in range(IDX_D)],
          [pltpu.SemaphoreType.DMA for _ in range(IDX_D)],
          [pltpu.SemaphoreType.DMA for _ in range(ROWS_D)],
          [pltpu.SemaphoreType.DMA for _ in range(ROWS_D)],
      ],
  )
  def agg(z_hbm, src_hbm, dst_hbm, out_hbm, src_v, dst_v, rows_v, acc_sh,
          sem_si, sem_di, sem_g, sem_s):
    c = lax.axis_index("c")
    s = lax.axis_index("s")
    w = s * NC + c
    r0 = s * rows_a

    @pl.when(s < NS - 1)
    def _():
      pltpu.sync_copy(z_hbm.at[pl.ds(r0, rows_a)],
                      acc_sh.at[pl.ds(r0, rows_a)])

    @pl.when(s == NS - 1)
    def _():
      pltpu.sync_copy(z_hbm.at[pl.ds(r0, rows_last)],
                      acc_sh.at[pl.ds(r0, rows_last)])

    plsc.subcore_barrier()

    def valid(slot):
      return (slot * nw + w) < n_chunks

    def stage_a(slot, ib):
      @pl.when(valid(slot))
      def _():
        base = (slot * nw + w) * CHUNK
        pltpu.async_copy(src_hbm.at[pl.ds(base, CHUNK)], src_v[ib], sem_si[ib])
        pltpu.async_copy(dst_hbm.at[pl.ds(base, CHUNK)], dst_v[ib], sem_di[ib])

    def stage_b(slot, ib, rb):
      @pl.when(valid(slot))
      def _():
        base = (slot * nw + w) * CHUNK
        pltpu.make_async_copy(src_hbm.at[pl.ds(base, CHUNK)], src_v[ib],
                              sem_si[ib]).wait()
        pltpu.async_copy(z_hbm.at[src_v[ib]], rows_v[rb], sem_g[rb])

    def stage_c(slot, ib, rb):
      @pl.when(valid(slot))
      def _():
        base = (slot * nw + w) * CHUNK
        pltpu.make_async_copy(z_hbm.at[src_v[ib]], rows_v[rb],
                              sem_g[rb]).wait()
        pltpu.make_async_copy(dst_hbm.at[pl.ds(base, CHUNK)], dst_v[ib],
                              sem_di[ib]).wait()
        pltpu.async_copy(rows_v[rb], acc_sh.at[dst_v[ib]], sem_s[rb],
                         add=True)

    def stage_w(slot, ib, rb):
      @pl.when(valid(slot))
      def _():
        pltpu.make_async_copy(rows_v[rb], acc_sh.at[dst_v[ib]],
                              sem_s[rb]).wait()

    def step(slot, k):
      if not isinstance(slot, int) or slot - 2 >= 0:
        stage_w(slot - 2, (k - 2) % IDX_D, (k - 2) % ROWS_D)
      stage_a(slot + LOOKA, (k + LOOKA) % IDX_D)
      stage_b(slot + 1, (k + 1) % IDX_D, (k + 1) % ROWS_D)
      stage_c(slot, k % IDX_D, k % ROWS_D)

    for p in range(LOOKA):
      stage_a(p, p % IDX_D)
    stage_b(0, 0, 0)
    for p in range(PEEL):
      step(p, p % IDX_D)

    n_main = (n_slots - 2 * PEEL) // PEEL

    def body(j, carry):
      s0 = PEEL + PEEL * j
      for k in range(PEEL):
        step(s0 + k, k % IDX_D)
      return carry

    lax.fori_loop(0, n_main, body, 0)
    for tail in range(PEEL + PEEL * n_main, n_slots):
      step(tail, tail % IDX_D)
    stage_w(n_slots - 2, (n_slots - 2) % IDX_D, (n_slots - 2) % ROWS_D)
    stage_w(n_slots - 1, (n_slots - 1) % IDX_D, (n_slots - 1) % ROWS_D)

    plsc.subcore_barrier()

    @pl.when(s < NS - 1)
    def _():
      pltpu.sync_copy(acc_sh.at[pl.ds(r0, rows_a)],
                      out_hbm.at[c, pl.ds(r0, rows_a)])

    @pl.when(s == NS - 1)
    def _():
      pltpu.sync_copy(acc_sh.at[pl.ds(r0, rows_last)],
                      out_hbm.at[c, pl.ds(r0, rows_last)])

  return agg(z, src, dst)


def _mlp_body(z_ref, p_ref, w1_ref, b1_ref, w2_ref, b2_ref,
              t_ref, sums_ref):
  i = pl.program_id(0)
  h = p_ref[0] + p_ref[1] - z_ref[...]
  h = jnp.maximum(
      jnp.dot(h, w1_ref[...], preferred_element_type=jnp.float32) + b1_ref[...],
      0.0)
  h = jnp.dot(h, w2_ref[...], preferred_element_type=jnp.float32) + b2_ref[...]
  t = jnp.maximum(h, 0.0)
  t_ref[...] = t

  @pl.when(i == 0)
  def _():
    sums_ref[...] = jnp.zeros_like(sums_ref)

  sums_ref[...] += jnp.concatenate(
      [jnp.sum(t, axis=0, keepdims=True),
       jnp.sum(t * t, axis=0, keepdims=True)], axis=0)


def _mlp_pass(z, partials, w1, b1, w2, b2):
  n, h = z.shape[0], w1.shape[1]
  nb = n // BN
  blk = lambda i: (i, 0)
  full = lambda i: (0, 0)
  return pl.pallas_call(
      _mlp_body,
      grid=(nb,),
      in_specs=[
          pl.BlockSpec((BN, z.shape[1]), blk),
          pl.BlockSpec((2, BN, z.shape[1]), lambda i: (0, i, 0)),
          pl.BlockSpec(w1.shape, full),
          pl.BlockSpec((1, h), full),
          pl.BlockSpec(w2.shape, full),
          pl.BlockSpec((1, h), full),
      ],
      out_specs=[
          pl.BlockSpec((BN, h), blk),
          pl.BlockSpec((2, h), full),
      ],
      out_shape=[
          jax.ShapeDtypeStruct((n, h), jnp.float32),
          jax.ShapeDtypeStruct((2, h), jnp.float32),
      ],
  )(z, partials, w1, b1.reshape(1, -1), w2, b2.reshape(1, -1))


def _bn_pool_core(t_ref, sums_ref, gamma_ref, beta_ref, batch_ref,
                  h_ref, g_ref, n_nodes, n_graphs):
  i = pl.program_id(0)
  inv_n = 1.0 / n_nodes
  mu = sums_ref[0:1, :] * inv_n
  var = jnp.maximum(sums_ref[1:2, :] * inv_n - mu * mu, 0.0)
  scale = gamma_ref[...] * lax.rsqrt(var + EPS)
  shift = beta_ref[...] - mu * scale
  h = t_ref[...] * scale + shift
  h_ref[...] = h

  bb = batch_ref[0]
  gi = lax.broadcasted_iota(jnp.int32, (n_graphs, h.shape[0]), 0)
  onehot = (gi == bb).astype(jnp.float32)

  @pl.when(i == 0)
  def _():
    g_ref[...] = jnp.zeros_like(g_ref)

  g_ref[...] += jnp.dot(onehot, h, preferred_element_type=jnp.float32)
  return h


def _bn_pool_pass(t, sums, gamma, beta, batch3d, n_graphs, col, prev=None):
  n, h = t.shape
  nb = n // BN
  in_specs = [
      pl.BlockSpec((BN, h), lambda i: (i, 0)),
      pl.BlockSpec((2, h), lambda i: (0, 0)),
      pl.BlockSpec((1, h), lambda i: (0, 0)),
      pl.BlockSpec((1, h), lambda i: (0, 0)),
      pl.BlockSpec((1, 1, BN), lambda i: (i, 0, 0)),
  ]
  args = [t, sums, gamma.reshape(1, -1), beta.reshape(1, -1), batch3d]
  kwargs = {}
  if prev is None:
    def body(t_ref, s_ref, ga_ref, be_ref, b_ref, h_ref, g_ref, hs_ref):
      hs_ref[...] = _bn_pool_core(t_ref, s_ref, ga_ref, be_ref, b_ref,
                                  h_ref, g_ref, n, n_graphs)
  else:
    def body(t_ref, s_ref, ga_ref, be_ref, b_ref, zi_ref, gi_ref,
             h_ref, g_ref):
      del zi_ref, gi_ref
      _bn_pool_core(t_ref, s_ref, ga_ref, be_ref, b_ref, h_ref, g_ref,
                    n, n_graphs)
    in_specs += [
        pl.BlockSpec((8, h), lambda i: (0, 0)),
        pl.BlockSpec((8, h), lambda i: (0, 0)),
    ]
    args += list(prev)
    kwargs["input_output_aliases"] = {5: 0, 6: 1}
  out_specs = [
      pl.BlockSpec((BN, h), lambda i: (i, col)),
      pl.BlockSpec((n_graphs, h), lambda i: (0, col)),
  ]
  out_shape = [
      jax.ShapeDtypeStruct((n, 2 * h), jnp.float32),
      jax.ShapeDtypeStruct((n_graphs, 2 * h), jnp.float32),
  ]
  if prev is None:
    out_specs.append(pl.BlockSpec((BN, h), lambda i: (i, 0)))
    out_shape.append(jax.ShapeDtypeStruct((n, h), jnp.float32))
  return pl.pallas_call(
      body,
      grid=(nb,),
      in_specs=in_specs,
      out_specs=out_specs,
      out_shape=out_shape,
      **kwargs,
  )(*args)


def kernel(x, edge_index, batch,
           W1_0, b1_0, W2_0, b2_0, gamma_0, beta_0,
           W1_1, b1_1, W2_1, b2_1, gamma_1, beta_1):
  n = x.shape[0]
  g_count = 512
  src = edge_index[0]
  dst = edge_index[1]
  batch3d = batch.reshape(n // BN, 1, BN)

  partials0 = _edge_partials(x, src, dst)
  t0, sums0 = _mlp_pass(x, partials0, W1_0, b1_0, W2_0, b2_0)
  zout, gout, h0 = _bn_pool_pass(t0, sums0, gamma_0, beta_0, batch3d,
                                 g_count, 0)
  partials1 = _edge_partials(h0, src, dst)
  t1, sums1 = _mlp_pass(h0, partials1, W1_1, b1_1, W2_1, b2_1)
  zout, gout = _bn_pool_pass(t1, sums1, gamma_1, beta_1, batch3d,
                             g_count, 1, prev=(zout, gout))
  return zout, gout
